# baseline (speedup 1.0000x reference)
import functools

import jax
import jax.numpy as jnp
from jax import lax
from jax.experimental import pallas as pl
from jax.experimental.pallas import tpu as pltpu
from jax.experimental.pallas import tpu_sc as plsc

_NC = 2
_NS = 16
_L = 16


def _build(B, F, V, D):
    NW = _NC * _NS
    BPW = B // NW
    C = 64
    NCH = BPW // C
    IPC = C * F
    NSUB = IPC // 128
    assert B % NW == 0 and BPW % C == 0 and IPC % 128 == 0

    mesh = plsc.VectorSubcoreMesh(core_axis_name="c", subcore_axis_name="s")

    @functools.partial(
        pl.kernel,
        mesh=mesh,
        compiler_params=pltpu.CompilerParams(
            needs_layout_passes=False, use_tc_tiling_on_sc=False),
        out_type=jax.ShapeDtypeStruct((B,), jnp.float32),
        scratch_types=[
            pltpu.VMEM((IPC,), jnp.int32),
            pltpu.VMEM((NSUB, 128), jnp.int32),
            pltpu.VMEM((IPC, _L), jnp.float32),
            pltpu.VMEM((IPC + _L,), jnp.float32),
            pltpu.VMEM((C,), jnp.float32),
            pltpu.VMEM((_L,), jnp.float32),
            pltpu.SemaphoreType.DMA,
        ],
    )
    def fm(x_hbm, emb_hbm, lin_hbm, lb_hbm, out_hbm,
           x_v, idx_v, rows_v, lin_v, out_v, lb_v, sem):
        cid = lax.axis_index("c")
        sid = lax.axis_index("s")
        wid = sid * _NC + cid
        base = wid * BPW

        pltpu.sync_copy(lb_hbm, lb_v.at[pl.ds(0, 1)])
        lb = lb_v[pl.ds(0, _L)][0]

        lane = lax.iota(jnp.int32, _L)
        m_tail = jnp.where(lane < (F - _L), 1.0, 0.0).astype(jnp.float32)
        m_last = lane == (_L - 1)
        zeros_i = jnp.zeros((_L,), jnp.int32)

        def chunk_body(ci, carry):
            cbase = base + ci * C
            pltpu.sync_copy(x_hbm.at[pl.ds(cbase * F, IPC)], x_v)

            def idx_body(r, carry2):
                for j in range(128 // _L):
                    p0 = r * 128 + j * _L
                    pos = p0 + lane
                    fld = lax.rem(pos, F)
                    idx_v[r, pl.ds(j * _L, _L)] = x_v[pl.ds(p0, _L)] + fld * V
                return carry2
            lax.fori_loop(0, NSUB, idx_body, 0)

            copies = []
            for j in range(NSUB):
                copies.append(pltpu.async_copy(
                    emb_hbm.at[idx_v.at[j]],
                    rows_v.at[pl.ds(j * 128, 128)], sem))
                copies.append(pltpu.async_copy(
                    lin_hbm.at[idx_v.at[j]],
                    lin_v.at[pl.ds(j * 128, 128)], sem))
            for cp in copies:
                cp.wait()

            def row_body(b, carry2):
                rbase = b * F
                acc = rows_v[rbase, :]
                acc2 = acc * acc
                for f in range(1, F):
                    r = rows_v[rbase + f, :]
                    acc = acc + r
                    acc2 = acc2 + r * r
                lv = lin_v[pl.ds(rbase, _L)] + lin_v[pl.ds(rbase + _L, _L)] * m_tail
                cs = plsc.cumsum(acc)
                cq = plsc.cumsum(acc2)
                cl = plsc.cumsum(lv)
                logit = cl + lb + 0.5 * (cs * cs - cq)
                sig = 1.0 / (1.0 + jnp.exp(-logit))
                plsc.store_scatter(out_v, [zeros_i + b], sig, mask=m_last)
                return carry2
            lax.fori_loop(0, C, row_body, 0)

            pltpu.sync_copy(out_v, out_hbm.at[pl.ds(cbase, C)])
            return carry
        lax.fori_loop(0, NCH, chunk_body, 0)

    return fm


def kernel(x, emb_table, lin_w, lin_b):
    B, F = x.shape
    D = emb_table.shape[1]
    V = emb_table.shape[0] // F
    fm = _build(B, F, V, D)
    out = fm(x.reshape(B * F).astype(jnp.int32), emb_table, lin_w, lin_b)
    return out.reshape(B, 1)
